# baseline (speedup 1.0000x reference)
import functools
import math

import jax
import jax.numpy as jnp
from jax import lax
from jax.experimental import pallas as pl
from jax.experimental.pallas import tpu as pltpu
from jax.experimental.pallas import tpu_sc as plsc

_GHM = 0.75
_NBINS = 10
_ROWS, _COLS = 16384, 1024
_TOTAL = float(_ROWS * _COLS)

_SC_ROWS = 11264
_TC_ROWS = _ROWS - _SC_ROWS

_BLK = 1024
_NSTEPS = _TC_ROWS // _BLK
_LOGIT_EDGES = [math.log(i / (_NBINS - i)) for i in range(1, _NBINS)]


def _tc_body(pred_ref, tgt_ref, out_ref, acc_ref):
    step = pl.program_id(0)

    @pl.when(step == 0)
    def _init():
        for i in range(20):
            acc_ref[i] = 0.0

    p = pred_ref[...]
    t = tgt_ref[...].astype(jnp.float32)
    sp = p * (1.0 - 2.0 * t)
    u = jnp.exp(-jnp.abs(sp))
    loss_e = jnp.maximum(sp, 0.0) + jnp.log(1.0 + u)

    acc_ref[19] += jnp.sum(loss_e)
    for i, le in enumerate(_LOGIT_EDGES):
        m = sp < le
        acc_ref[i] += jnp.sum(m.astype(jnp.float32))
        acc_ref[9 + i] += jnp.sum(jnp.where(m, loss_e, 0.0))

    @pl.when(step == _NSTEPS - 1)
    def _fini():
        for i in range(20):
            out_ref[i] = acc_ref[i]


def _tc_partials(pred, target):
    return pl.pallas_call(
        _tc_body,
        grid=(_NSTEPS,),
        in_specs=[
            pl.BlockSpec((_BLK, _COLS), lambda i: (i, 0)),
            pl.BlockSpec((_BLK, _COLS), lambda i: (i, 0)),
        ],
        out_specs=pl.BlockSpec(memory_space=pltpu.SMEM),
        out_shape=jax.ShapeDtypeStruct((20,), jnp.float32),
        scratch_shapes=[pltpu.SMEM((20,), jnp.float32)],
    )(pred, target)


_NC, _NS = 2, 16
_NW = _NC * _NS
_WROWS = _SC_ROWS // _NW
_RCH = 16
_NCH = _WROWS // _RCH
_ABS_EDGES = [math.log(e / (1.0 - e)) for e in (0.6, 0.7, 0.8, 0.9)]
_LOG1P_C = [3.910905554960209e-08, 0.9999936302585147, -0.49982549864347925,
            0.33144665224343317, -0.23943337074600235, 0.16499812983410006,
            -0.09229041738055285, 0.03426459995555095, -0.006006605050865348]

_sc_mesh = plsc.VectorSubcoreMesh(core_axis_name="c", subcore_axis_name="s")


@functools.partial(
    pl.kernel,
    out_type=(
        jax.ShapeDtypeStruct((_NW * 16, _NBINS), jnp.float32),
        jax.ShapeDtypeStruct((_NW * 16, _NBINS), jnp.float32),
    ),
    mesh=_sc_mesh,
    compiler_params=pltpu.CompilerParams(needs_layout_passes=False),
    scratch_types=[
        pltpu.VMEM((2, _RCH, _COLS), jnp.float32),
        pltpu.VMEM((2, _RCH, _COLS), jnp.int32),
        pltpu.VMEM((16, _NBINS), jnp.float32),
        pltpu.VMEM((16, _NBINS), jnp.float32),
        pltpu.SemaphoreType.DMA((2,)),
    ],
)
def _sc_partials(pred_hbm, tgt_hbm, cnt_hbm, sum_hbm, pbuf, tbuf, cnt_v, sum_v, sem):
    wid = lax.axis_index("s") * _NC + lax.axis_index("c")
    row0 = _TC_ROWS + wid * _WROWS
    lane = lax.iota(jnp.int32, 16)
    ones = jnp.ones((16,), jnp.float32)
    zeros16 = jnp.zeros((16,), jnp.float32)
    for j in range(_NBINS):
        colj = jnp.full((16,), j, jnp.int32)
        plsc.store_scatter(cnt_v, [lane, colj], zeros16)
        plsc.store_scatter(sum_v, [lane, colj], zeros16)

    def start(ci, slot):
        r = row0 + ci * _RCH
        return (
            pltpu.async_copy(pred_hbm.at[pl.ds(r, _RCH)], pbuf.at[slot], sem.at[slot]),
            pltpu.async_copy(tgt_hbm.at[pl.ds(r, _RCH)], tbuf.at[slot], sem.at[slot]),
        )

    pending = start(0, 0)
    for ci in range(_NCH):
        slot = ci % 2
        for d in pending:
            d.wait()
        if ci + 1 < _NCH:
            pending = start(ci + 1, 1 - slot)

        @functools.partial(plsc.parallel_loop, 0, _RCH * 16, unroll=2)
        def _chunk_loop(j):
            rr = lax.shift_right_logical(j, 4)
            cb = pl.multiple_of(lax.shift_left(jnp.bitwise_and(j, 15), 6), 64)
            for k in range(4):
                cc = pl.multiple_of(cb + 16 * k, 16)
                p = pbuf[slot, rr, pl.ds(cc, 16)]
                tf = tbuf[slot, rr, pl.ds(cc, 16)].astype(jnp.float32)
                sp = p * (1.0 - 2.0 * tf)
                a = jnp.abs(sp)
                u = jnp.exp(-a)
                q = jnp.zeros((16,), jnp.int32)
                for m in _ABS_EDGES:
                    q = q + jnp.where(a >= m, 1, 0)
                b = jnp.where(sp >= 0.0, q + 5, 4 - q)
                h = jnp.full((16,), _LOG1P_C[8], jnp.float32)
                for c in reversed(_LOG1P_C[:8]):
                    h = h * u + c
                loss_e = jnp.maximum(sp, 0.0) + h
                plsc.addupdate_scatter(cnt_v, [lane, b], ones)
                plsc.addupdate_scatter(sum_v, [lane, b], loss_e)

    pltpu.sync_copy(cnt_v, cnt_hbm.at[pl.ds(wid * 16, 16)])
    pltpu.sync_copy(sum_v, sum_hbm.at[pl.ds(wid * 16, 16)])


def _combine_body(acc_ref, cnt_ref, sum_ref, out_ref):
    cvec = jnp.sum(cnt_ref[...], axis=0, keepdims=True)
    svec = jnp.sum(sum_ref[...], axis=0, keepdims=True)
    iota = lax.broadcasted_iota(jnp.int32, (1, _NBINS), 1)
    lt = [0.0] + [acc_ref[i] for i in range(9)] + [float(_TC_ROWS * _COLS)]
    wlt = [0.0] + [acc_ref[9 + i] for i in range(9)] + [acc_ref[19]]
    for i in range(_NBINS):
        cvec = cvec + jnp.where(iota == i, lt[i + 1] - lt[i], 0.0)
        svec = svec + jnp.where(iota == i, wlt[i + 1] - wlt[i], 0.0)
    out_ref[0] = jnp.sum(svec / jnp.maximum(cvec, 1.0)) * (_GHM / 10.0)


def _combine(tc_acc, sc_cnt, sc_sum):
    return pl.pallas_call(
        _combine_body,
        in_specs=[
            pl.BlockSpec(memory_space=pltpu.SMEM),
            pl.BlockSpec((_NW * 16, _NBINS), lambda: (0, 0)),
            pl.BlockSpec((_NW * 16, _NBINS), lambda: (0, 0)),
        ],
        out_specs=pl.BlockSpec(memory_space=pltpu.SMEM),
        out_shape=jax.ShapeDtypeStruct((1,), jnp.float32),
    )(tc_acc, sc_cnt, sc_sum)


def kernel(pred, target):
    sc_cnt, sc_sum = _sc_partials(pred, target)
    tc_acc = _tc_partials(pred, target)
    return _combine(tc_acc, sc_cnt, sc_sum).reshape(())


# speedup vs baseline: 1.0362x; 1.0362x over previous
import functools
import math

import jax
import jax.numpy as jnp
from jax import lax
from jax.experimental import pallas as pl
from jax.experimental.pallas import tpu as pltpu
from jax.experimental.pallas import tpu_sc as plsc

_GHM = 0.75
_NBINS = 10
_ROWS, _COLS = 16384, 1024
_TOTAL = float(_ROWS * _COLS)

_SC_ROWS = 12288
_TC_ROWS = _ROWS - _SC_ROWS

_BLK = 1024
_NSTEPS = _TC_ROWS // _BLK
_LOGIT_EDGES = [math.log(i / (_NBINS - i)) for i in range(1, _NBINS)]


def _tc_body(pred_ref, tgt_ref, out_ref, acc_ref):
    step = pl.program_id(0)

    @pl.when(step == 0)
    def _init():
        for i in range(20):
            acc_ref[i] = 0.0

    p = pred_ref[...]
    t = tgt_ref[...].astype(jnp.float32)
    sp = p * (1.0 - 2.0 * t)
    u = jnp.exp(-jnp.abs(sp))
    loss_e = jnp.maximum(sp, 0.0) + jnp.log(1.0 + u)

    acc_ref[19] += jnp.sum(loss_e)
    for i, le in enumerate(_LOGIT_EDGES):
        m = sp < le
        acc_ref[i] += jnp.sum(m.astype(jnp.float32))
        acc_ref[9 + i] += jnp.sum(jnp.where(m, loss_e, 0.0))

    @pl.when(step == _NSTEPS - 1)
    def _fini():
        for i in range(20):
            out_ref[i] = acc_ref[i]


def _tc_partials(pred, target):
    return pl.pallas_call(
        _tc_body,
        grid=(_NSTEPS,),
        in_specs=[
            pl.BlockSpec((_BLK, _COLS), lambda i: (i, 0)),
            pl.BlockSpec((_BLK, _COLS), lambda i: (i, 0)),
        ],
        out_specs=pl.BlockSpec(memory_space=pltpu.SMEM),
        out_shape=jax.ShapeDtypeStruct((20,), jnp.float32),
        scratch_shapes=[pltpu.SMEM((20,), jnp.float32)],
    )(pred, target)


_NC, _NS = 2, 16
_NW = _NC * _NS
_WROWS = _SC_ROWS // _NW
_RCH = 16
_NCH = _WROWS // _RCH
_ABS_EDGES = [math.log(e / (1.0 - e)) for e in (0.6, 0.7, 0.8, 0.9)]
_LOG1P_C = [3.910905554960209e-08, 0.9999936302585147, -0.49982549864347925,
            0.33144665224343317, -0.23943337074600235, 0.16499812983410006,
            -0.09229041738055285, 0.03426459995555095, -0.006006605050865348]

_sc_mesh = plsc.VectorSubcoreMesh(core_axis_name="c", subcore_axis_name="s")


@functools.partial(
    pl.kernel,
    out_type=(
        jax.ShapeDtypeStruct((_NW * 16, _NBINS), jnp.float32),
        jax.ShapeDtypeStruct((_NW * 16, _NBINS), jnp.float32),
    ),
    mesh=_sc_mesh,
    compiler_params=pltpu.CompilerParams(needs_layout_passes=False),
    scratch_types=[
        pltpu.VMEM((2, _RCH, _COLS), jnp.float32),
        pltpu.VMEM((2, _RCH, _COLS), jnp.int32),
        pltpu.VMEM((16, _NBINS), jnp.float32),
        pltpu.VMEM((16, _NBINS), jnp.float32),
        pltpu.SemaphoreType.DMA((2,)),
    ],
)
def _sc_partials(pred_hbm, tgt_hbm, cnt_hbm, sum_hbm, pbuf, tbuf, cnt_v, sum_v, sem):
    wid = lax.axis_index("s") * _NC + lax.axis_index("c")
    row0 = _TC_ROWS + wid * _WROWS
    lane = lax.iota(jnp.int32, 16)
    ones = jnp.ones((16,), jnp.float32)
    zeros16 = jnp.zeros((16,), jnp.float32)
    for j in range(_NBINS):
        colj = jnp.full((16,), j, jnp.int32)
        plsc.store_scatter(cnt_v, [lane, colj], zeros16)
        plsc.store_scatter(sum_v, [lane, colj], zeros16)

    def start(ci, slot):
        r = row0 + ci * _RCH
        return (
            pltpu.async_copy(pred_hbm.at[pl.ds(r, _RCH)], pbuf.at[slot], sem.at[slot]),
            pltpu.async_copy(tgt_hbm.at[pl.ds(r, _RCH)], tbuf.at[slot], sem.at[slot]),
        )

    pending = start(0, 0)
    for ci in range(_NCH):
        slot = ci % 2
        for d in pending:
            d.wait()
        if ci + 1 < _NCH:
            pending = start(ci + 1, 1 - slot)

        @functools.partial(plsc.parallel_loop, 0, _RCH * 16, unroll=2)
        def _chunk_loop(j):
            rr = lax.shift_right_logical(j, 4)
            cb = pl.multiple_of(lax.shift_left(jnp.bitwise_and(j, 15), 6), 64)
            for k in range(4):
                cc = pl.multiple_of(cb + 16 * k, 16)
                p = pbuf[slot, rr, pl.ds(cc, 16)]
                tf = tbuf[slot, rr, pl.ds(cc, 16)].astype(jnp.float32)
                sp = p * (1.0 - 2.0 * tf)
                a = jnp.abs(sp)
                u = jnp.exp(-a)
                q = jnp.zeros((16,), jnp.int32)
                for m in _ABS_EDGES:
                    q = q + jnp.where(a >= m, 1, 0)
                b = jnp.where(sp >= 0.0, q + 5, 4 - q)
                h = jnp.full((16,), _LOG1P_C[8], jnp.float32)
                for c in reversed(_LOG1P_C[:8]):
                    h = h * u + c
                loss_e = jnp.maximum(sp, 0.0) + h
                plsc.addupdate_scatter(cnt_v, [lane, b], ones)
                plsc.addupdate_scatter(sum_v, [lane, b], loss_e)

    pltpu.sync_copy(cnt_v, cnt_hbm.at[pl.ds(wid * 16, 16)])
    pltpu.sync_copy(sum_v, sum_hbm.at[pl.ds(wid * 16, 16)])


def _combine_body(acc_ref, cnt_ref, sum_ref, out_ref):
    cvec = jnp.sum(cnt_ref[...], axis=0, keepdims=True)
    svec = jnp.sum(sum_ref[...], axis=0, keepdims=True)
    iota = lax.broadcasted_iota(jnp.int32, (1, _NBINS), 1)
    lt = [0.0] + [acc_ref[i] for i in range(9)] + [float(_TC_ROWS * _COLS)]
    wlt = [0.0] + [acc_ref[9 + i] for i in range(9)] + [acc_ref[19]]
    for i in range(_NBINS):
        cvec = cvec + jnp.where(iota == i, lt[i + 1] - lt[i], 0.0)
        svec = svec + jnp.where(iota == i, wlt[i + 1] - wlt[i], 0.0)
    out_ref[0] = jnp.sum(svec / jnp.maximum(cvec, 1.0)) * (_GHM / 10.0)


def _combine(tc_acc, sc_cnt, sc_sum):
    return pl.pallas_call(
        _combine_body,
        in_specs=[
            pl.BlockSpec(memory_space=pltpu.SMEM),
            pl.BlockSpec((_NW * 16, _NBINS), lambda: (0, 0)),
            pl.BlockSpec((_NW * 16, _NBINS), lambda: (0, 0)),
        ],
        out_specs=pl.BlockSpec(memory_space=pltpu.SMEM),
        out_shape=jax.ShapeDtypeStruct((1,), jnp.float32),
    )(tc_acc, sc_cnt, sc_sum)


def kernel(pred, target):
    sc_cnt, sc_sum = _sc_partials(pred, target)
    tc_acc = _tc_partials(pred, target)
    return _combine(tc_acc, sc_cnt, sc_sum).reshape(())
